# baseline (speedup 1.0000x reference)
import functools

import jax
import jax.numpy as jnp
from jax import lax
from jax.experimental import pallas as pl
from jax.experimental.pallas import tpu as pltpu
from jax.experimental.pallas import tpu_sc as plsc

NC, NS, L = 2, 16, 16
NW = NC * NS
N = 10000
NPAD = 10240
E = 320000
F = 128
KCH = 80
EPAD = 327680
ROWS = EPAD // KCH
RPW = ROWS // NW
NPT = NPAD // NS
EPS = 1e-5
OUT_F = 16

_mesh = plsc.VectorSubcoreMesh(core_axis_name="c", subcore_axis_name="s")
_sc_params = pltpu.CompilerParams(needs_layout_passes=False)

_f32 = jnp.float32


def _ids():
    c = lax.axis_index("c")
    s = lax.axis_index("s")
    return c, s, s * NC + c


def _deg_body(src_hbm, ew_hbm, out0_hbm, out1_hbm, srcb, ewb, zb, dacc):
    c, s, wid = _ids()
    tbase = s * NPT

    @pl.loop(0, NPT // L)
    def _zero(i):
        zb[pl.ds(i * L, L)] = jnp.zeros((L,), _f32)

    pltpu.sync_copy(zb, dacc.at[pl.ds(tbase, NPT)])
    plsc.subcore_barrier()

    pltpu.sync_copy(src_hbm.at[pl.ds(wid * RPW, RPW)], srcb)
    pltpu.sync_copy(ew_hbm.at[pl.ds(wid * RPW, RPW)], ewb)

    @pl.loop(0, RPW)
    def _chunk(jj):
        pltpu.sync_copy(ewb.at[jj], dacc.at[srcb.at[jj]], add=True)

    plsc.subcore_barrier()

    @pl.when(c == 0)
    def _w0():
        pltpu.sync_copy(dacc.at[pl.ds(tbase, NPT)], out0_hbm.at[pl.ds(tbase, NPT)])

    @pl.when(c == 1)
    def _w1():
        pltpu.sync_copy(dacc.at[pl.ds(tbase, NPT)], out1_hbm.at[pl.ds(tbase, NPT)])


_deg_call = pl.kernel(
    _deg_body,
    out_type=[jax.ShapeDtypeStruct((NPAD,), _f32),
              jax.ShapeDtypeStruct((NPAD,), _f32)],
    mesh=_mesh,
    compiler_params=_sc_params,
    scratch_types=[
        pltpu.VMEM((RPW, KCH), jnp.int32),
        pltpu.VMEM((RPW, KCH), _f32),
        pltpu.VMEM((NPT,), _f32),
        pltpu.VMEM_SHARED((NPAD,), _f32),
    ],
)


def _dinv_body(d0_ref, d1_ref, o_ref):
    d = d0_ref[...] + d1_ref[...]
    o_ref[...] = jnp.where(d > 0, lax.rsqrt(jnp.maximum(d, 1e-30)), 0.0)


def _dinv_call(d0, d1):
    r = pl.pallas_call(
        _dinv_body,
        out_shape=jax.ShapeDtypeStruct((NPAD // F, F), _f32),
    )(d0.reshape(NPAD // F, F), d1.reshape(NPAD // F, F))
    return r.reshape(NPAD)


def _wn_body(dinv_hbm, src_hbm, dst_hbm, ew_hbm, wn_hbm,
             dv, srcb, dstb, ewb, wnb):
    c, s, wid = _ids()
    pltpu.sync_copy(dinv_hbm, dv)

    pltpu.sync_copy(src_hbm.at[pl.ds(wid * RPW, RPW)], srcb)
    pltpu.sync_copy(dst_hbm.at[pl.ds(wid * RPW, RPW)], dstb)
    pltpu.sync_copy(ew_hbm.at[pl.ds(wid * RPW, RPW)], ewb)

    @pl.loop(0, RPW)
    def _row(jj):
        for q in range(KCH // L):
            sl = pl.ds(q * L, L)
            g1 = plsc.load_gather(dv, [srcb[jj, sl]])
            g2 = plsc.load_gather(dv, [dstb[jj, sl]])
            wnb[jj, sl] = -(g1 * ewb[jj, sl] * g2)

    pltpu.sync_copy(wnb, wn_hbm.at[pl.ds(wid * RPW, RPW)])


_wn_call = pl.kernel(
    _wn_body,
    out_type=jax.ShapeDtypeStruct((ROWS, KCH), _f32),
    mesh=_mesh,
    compiler_params=_sc_params,
    scratch_types=[
        pltpu.VMEM((NPAD,), _f32),
        pltpu.VMEM((RPW, KCH), jnp.int32),
        pltpu.VMEM((RPW, KCH), jnp.int32),
        pltpu.VMEM((RPW, KCH), _f32),
        pltpu.VMEM((RPW, KCH), _f32),
    ],
)


SLAB = 16


def _scale_chunk(wnb, rbb, jj):
    @pl.loop(0, KCH // L)
    def _scale(g):
        w16 = wnb[jj, pl.ds(g * L, L)]
        for e in range(L):
            wv = w16[e]
            row = g * L + e
            for q in range(F // L):
                sl = pl.ds(q * L, L)
                rbb[row, sl] = rbb[row, sl] * wv


_mesh1 = plsc.VectorSubcoreMesh(core_axis_name="c", subcore_axis_name="s",
                                num_cores=1)
RPW1C = ROWS // NS


def _spmm_edges(h_hbm, src_hbm, dst_hbm, wn_hbm, srcb, dstb, wnb, rb, acc,
                sems, base, npieces):
    @pl.loop(0, npieces)
    def _piece(p):
        poff = pl.multiple_of(base + p * SLAB, SLAB)
        pltpu.sync_copy(src_hbm.at[pl.ds(poff, SLAB)], srcb)
        pltpu.sync_copy(dst_hbm.at[pl.ds(poff, SLAB)], dstb)
        pltpu.sync_copy(wn_hbm.at[pl.ds(poff, SLAB)], wnb)

        pltpu.async_copy(h_hbm.at[srcb.at[0]], rb.at[0], sems[0])
        pltpu.async_copy(h_hbm.at[srcb.at[1]], rb.at[1], sems[1])

        @pl.loop(0, SLAB, step=2)
        def _pair(jj):
            for b in range(2):
                jb = jj + b
                pltpu.make_async_copy(h_hbm.at[srcb.at[jb]], rb.at[b], sems[b]).wait()
                _scale_chunk(wnb, rb.at[b], jb)
                pltpu.sync_copy(rb.at[b], acc.at[dstb.at[jb]], add=True)

                @pl.when(jb + 2 < SLAB)
                def _next():
                    pltpu.async_copy(h_hbm.at[srcb.at[jb + 2]], rb.at[b], sems[b])


def _spmm_body(h_hbm, src_hbm, dst_hbm, wn_hbm, out_hbm,
               srcb, dstb, wnb, rb, zb, acc, gs0, gs1):
    c = lax.axis_index("c")
    s = lax.axis_index("s")
    tbase = s * NPT
    sems = (gs0, gs1)

    @pl.when(c == 0)
    def _core0():
        @pl.loop(0, SLAB)
        def _zero(r):
            for q in range(F // L):
                zb[r, pl.ds(q * L, L)] = jnp.zeros((L,), _f32)

        @pl.loop(0, NPT // SLAB)
        def _zc(b):
            pltpu.sync_copy(zb, acc.at[pl.ds(tbase + b * SLAB, SLAB)])

        plsc.subcore_barrier()

        _spmm_edges(h_hbm, src_hbm, dst_hbm, wn_hbm, srcb, dstb, wnb, rb, acc,
                    sems, s * RPW1C, RPW1C // SLAB)

        plsc.subcore_barrier()
        pltpu.sync_copy(acc.at[pl.ds(tbase, NPT)], out_hbm.at[pl.ds(tbase, NPT)])


_spmm_call = pl.kernel(
    _spmm_body,
    out_type=jax.ShapeDtypeStruct((NPAD, F), _f32),
    mesh=_mesh,
    compiler_params=_sc_params,
    scratch_types=[
        pltpu.VMEM((SLAB, KCH), jnp.int32),
        pltpu.VMEM((SLAB, KCH), jnp.int32),
        pltpu.VMEM((SLAB, KCH), _f32),
        pltpu.VMEM((2, KCH, F), _f32),
        pltpu.VMEM((SLAB, F), _f32),
        pltpu.VMEM_SHARED((NPAD, F), _f32),
        pltpu.SemaphoreType.DMA,
        pltpu.SemaphoreType.DMA,
    ],
)


RB = 1280


_BN_INV = 1.0 / (1.0 + EPS) ** 0.5


def _comb_mid_body(h_ref, u1_ref, u2_ref, w_ref, b_ref, bnw_ref, bnb_ref, o_ref):
    h = h_ref[...]
    u2 = u2_ref[...]
    w0, w1, w2 = w_ref[0], w_ref[1], w_ref[2]
    z = jnp.dot(h, w0 - w2, preferred_element_type=_f32)
    z = z + jnp.dot(u1_ref[...], w1, preferred_element_type=_f32)
    z = z + 2.0 * jnp.dot(u2, w2, preferred_element_type=_f32)
    z = z + b_ref[...]
    o_ref[...] = jnp.maximum(z, 0.0) * (bnw_ref[...] * _BN_INV) + bnb_ref[...]


def _comb_mid(h, u1, u2, W, b, bnw, bnb):
    return pl.pallas_call(
        _comb_mid_body,
        grid=(NPAD // RB,),
        in_specs=[
            pl.BlockSpec((RB, F), lambda i: (i, 0)),
            pl.BlockSpec((RB, F), lambda i: (i, 0)),
            pl.BlockSpec((RB, F), lambda i: (i, 0)),
            pl.BlockSpec((3, F, F), lambda i: (0, 0, 0)),
            pl.BlockSpec((1, F), lambda i: (0, 0)),
            pl.BlockSpec((1, F), lambda i: (0, 0)),
            pl.BlockSpec((1, F), lambda i: (0, 0)),
        ],
        out_specs=pl.BlockSpec((RB, F), lambda i: (i, 0)),
        out_shape=jax.ShapeDtypeStruct((NPAD, F), _f32),
    )(h, u1, u2, W, b, bnw, bnb)


def _comb_fin_body(h_ref, u1_ref, u2_ref, w_ref, b_ref, bnw_ref, bnb_ref,
                   lw_ref, lb_ref, o_ref):
    h = h_ref[...]
    u2 = u2_ref[...]
    w0, w1, w2 = w_ref[0], w_ref[1], w_ref[2]
    z = jnp.dot(h, w0 - w2, preferred_element_type=_f32)
    z = z + jnp.dot(u1_ref[...], w1, preferred_element_type=_f32)
    z = z + 2.0 * jnp.dot(u2, w2, preferred_element_type=_f32)
    z = z + b_ref[...]
    h2 = jnp.maximum(z, 0.0) * (bnw_ref[...] * _BN_INV) + bnb_ref[...]
    o_ref[...] = jnp.dot(h2, lw_ref[...], preferred_element_type=_f32) + lb_ref[...]


def _comb_fin(h, u1, u2, W, b, bnw, bnb, lwT, lb):
    return pl.pallas_call(
        _comb_fin_body,
        grid=(NPAD // RB,),
        in_specs=[
            pl.BlockSpec((RB, F), lambda i: (i, 0)),
            pl.BlockSpec((RB, F), lambda i: (i, 0)),
            pl.BlockSpec((RB, F), lambda i: (i, 0)),
            pl.BlockSpec((3, F, F), lambda i: (0, 0, 0)),
            pl.BlockSpec((1, F), lambda i: (0, 0)),
            pl.BlockSpec((1, F), lambda i: (0, 0)),
            pl.BlockSpec((1, F), lambda i: (0, 0)),
            pl.BlockSpec((F, OUT_F), lambda i: (0, 0)),
            pl.BlockSpec((1, OUT_F), lambda i: (0, 0)),
        ],
        out_specs=pl.BlockSpec((RB, OUT_F), lambda i: (i, 0)),
        out_shape=jax.ShapeDtypeStruct((NPAD, OUT_F), _f32),
    )(h, u1, u2, W, b, bnw, bnb, lwT, lb)


def kernel(x, edge_index, edge_weight, W1, b1, bn1_w, bn1_b, W2, b2, bn2_w, bn2_b, lin_w, lin_b):
    src = jnp.pad(edge_index[0], (0, EPAD - E)).reshape(ROWS, KCH)
    dst = jnp.pad(edge_index[1], (0, EPAD - E)).reshape(ROWS, KCH)
    ew = jnp.pad(edge_weight, (0, EPAD - E)).reshape(ROWS, KCH)
    xp = jnp.pad(x, ((0, NPAD - N), (0, 0)))

    deg0, deg1 = _deg_call(src, ew)
    dinv = _dinv_call(deg0, deg1)
    wn = _wn_call(dinv, src, dst, ew)

    def spmm2(h):
        u1 = _spmm_call(h, src, dst, wn)
        u2 = _spmm_call(u1, src, dst, wn)
        return u1, u2

    u1, u2 = spmm2(xp)
    h2 = _comb_mid(xp, u1, u2, W1, b1.reshape(1, F), bn1_w.reshape(1, F),
                   bn1_b.reshape(1, F))
    v1, v2 = spmm2(h2)
    out = _comb_fin(h2, v1, v2, W2, b2.reshape(1, F), bn2_w.reshape(1, F),
                    bn2_b.reshape(1, F), lin_w.T, lin_b.reshape(1, OUT_F))
    return out[:N]


# speedup vs baseline: 1.3518x; 1.3518x over previous
import functools

import jax
import jax.numpy as jnp
from jax import lax
from jax.experimental import pallas as pl
from jax.experimental.pallas import tpu as pltpu
from jax.experimental.pallas import tpu_sc as plsc

NC, NS, L = 2, 16, 16
NW = NC * NS
N = 10000
NPAD = 10240
E = 320000
F = 128
KCH = 80
EPAD = 327680
ROWS = EPAD // KCH
RPW = ROWS // NW
NPT = NPAD // NS
EPS = 1e-5
OUT_F = 16

_mesh = plsc.VectorSubcoreMesh(core_axis_name="c", subcore_axis_name="s")
_sc_params = pltpu.CompilerParams(needs_layout_passes=False)

_f32 = jnp.float32


def _ids():
    c = lax.axis_index("c")
    s = lax.axis_index("s")
    return c, s, s * NC + c


def _deg_body(src_hbm, ew_hbm, out0_hbm, out1_hbm, srcb, ewb, zb, dacc):
    c, s, wid = _ids()
    tbase = s * NPT

    @pl.loop(0, NPT // L)
    def _zero(i):
        zb[pl.ds(i * L, L)] = jnp.zeros((L,), _f32)

    pltpu.sync_copy(zb, dacc.at[pl.ds(tbase, NPT)])
    plsc.subcore_barrier()

    pltpu.sync_copy(src_hbm.at[pl.ds(wid * RPW, RPW)], srcb)
    pltpu.sync_copy(ew_hbm.at[pl.ds(wid * RPW, RPW)], ewb)

    @pl.loop(0, RPW)
    def _chunk(jj):
        pltpu.sync_copy(ewb.at[jj], dacc.at[srcb.at[jj]], add=True)

    plsc.subcore_barrier()

    @pl.when(c == 0)
    def _w0():
        pltpu.sync_copy(dacc.at[pl.ds(tbase, NPT)], out0_hbm.at[pl.ds(tbase, NPT)])

    @pl.when(c == 1)
    def _w1():
        pltpu.sync_copy(dacc.at[pl.ds(tbase, NPT)], out1_hbm.at[pl.ds(tbase, NPT)])


_deg_call = pl.kernel(
    _deg_body,
    out_type=[jax.ShapeDtypeStruct((NPAD,), _f32),
              jax.ShapeDtypeStruct((NPAD,), _f32)],
    mesh=_mesh,
    compiler_params=_sc_params,
    scratch_types=[
        pltpu.VMEM((RPW, KCH), jnp.int32),
        pltpu.VMEM((RPW, KCH), _f32),
        pltpu.VMEM((NPT,), _f32),
        pltpu.VMEM_SHARED((NPAD,), _f32),
    ],
)


def _dinv_body(d0_ref, d1_ref, o_ref):
    d = d0_ref[...] + d1_ref[...]
    o_ref[...] = jnp.where(d > 0, lax.rsqrt(jnp.maximum(d, 1e-30)), 0.0)


def _dinv_call(d0, d1):
    r = pl.pallas_call(
        _dinv_body,
        out_shape=jax.ShapeDtypeStruct((NPAD // F, F), _f32),
    )(d0.reshape(NPAD // F, F), d1.reshape(NPAD // F, F))
    return r.reshape(NPAD)


def _wn_body(dinv_hbm, src_hbm, dst_hbm, ew_hbm, wn_hbm,
             dv, srcb, dstb, ewb, wnb):
    c, s, wid = _ids()
    pltpu.sync_copy(dinv_hbm, dv)

    pltpu.sync_copy(src_hbm.at[pl.ds(wid * RPW, RPW)], srcb)
    pltpu.sync_copy(dst_hbm.at[pl.ds(wid * RPW, RPW)], dstb)
    pltpu.sync_copy(ew_hbm.at[pl.ds(wid * RPW, RPW)], ewb)

    @pl.loop(0, RPW)
    def _row(jj):
        for q in range(KCH // L):
            sl = pl.ds(q * L, L)
            g1 = plsc.load_gather(dv, [srcb[jj, sl]])
            g2 = plsc.load_gather(dv, [dstb[jj, sl]])
            wnb[jj, sl] = -(g1 * ewb[jj, sl] * g2)

    pltpu.sync_copy(wnb, wn_hbm.at[pl.ds(wid * RPW, RPW)])


_wn_call = pl.kernel(
    _wn_body,
    out_type=jax.ShapeDtypeStruct((ROWS, KCH), _f32),
    mesh=_mesh,
    compiler_params=_sc_params,
    scratch_types=[
        pltpu.VMEM((NPAD,), _f32),
        pltpu.VMEM((RPW, KCH), jnp.int32),
        pltpu.VMEM((RPW, KCH), jnp.int32),
        pltpu.VMEM((RPW, KCH), _f32),
        pltpu.VMEM((RPW, KCH), _f32),
    ],
)


SLAB = 16


def _scale_chunk(wnb, rbb, jj):
    @pl.loop(0, KCH // L)
    def _scale(g):
        w16 = wnb[jj, pl.ds(g * L, L)]
        for e in range(L):
            wv = w16[e]
            row = g * L + e
            for q in range(F // L):
                sl = pl.ds(q * L, L)
                rbb[row, sl] = rbb[row, sl] * wv


RPW0 = 208
RPW1 = 2 * RPW - RPW0


def _spmm_edges(h_hbm, src_hbm, dst_hbm, wn_hbm, srcb, dstb, wnb, rb, acc,
                sems, base, npieces):
    @pl.loop(0, npieces)
    def _piece(p):
        poff = pl.multiple_of(base + p * SLAB, SLAB)
        pltpu.sync_copy(src_hbm.at[pl.ds(poff, SLAB)], srcb)
        pltpu.sync_copy(dst_hbm.at[pl.ds(poff, SLAB)], dstb)
        pltpu.sync_copy(wn_hbm.at[pl.ds(poff, SLAB)], wnb)

        pltpu.async_copy(h_hbm.at[srcb.at[0]], rb.at[0], sems[0])
        pltpu.async_copy(h_hbm.at[srcb.at[1]], rb.at[1], sems[1])

        @pl.loop(0, SLAB, step=2)
        def _pair(jj):
            for b in range(2):
                jb = jj + b
                pltpu.make_async_copy(h_hbm.at[srcb.at[jb]], rb.at[b], sems[b]).wait()
                _scale_chunk(wnb, rb.at[b], jb)
                pltpu.sync_copy(rb.at[b], acc.at[dstb.at[jb]], add=True)

                @pl.when(jb + 2 < SLAB)
                def _next():
                    pltpu.async_copy(h_hbm.at[srcb.at[jb + 2]], rb.at[b], sems[b])


def _spmm_body(h_hbm, src_hbm, dst_hbm, wn_hbm, out_hbm,
               srcb, dstb, wnb, rb, zb, acc, gs0, gs1):
    c, s, wid = _ids()
    tbase = s * NPT
    sems = (gs0, gs1)

    @pl.loop(0, SLAB)
    def _zero(r):
        for q in range(F // L):
            zb[r, pl.ds(q * L, L)] = jnp.zeros((L,), _f32)

    @pl.loop(0, NPT // SLAB)
    def _zc(b):
        pltpu.sync_copy(zb, acc.at[pl.ds(tbase + b * SLAB, SLAB)])

    plsc.subcore_barrier()

    args = (h_hbm, src_hbm, dst_hbm, wn_hbm, srcb, dstb, wnb, rb, acc, sems)

    @pl.when(c == 0)
    def _c0():
        _spmm_edges(*args, s * RPW0, RPW0 // SLAB)

    @pl.when(c == 1)
    def _c1():
        _spmm_edges(*args, NS * RPW0 + s * RPW1, RPW1 // SLAB)

    plsc.subcore_barrier()
    pltpu.sync_copy(acc.at[pl.ds(tbase, NPT)], out_hbm.at[c, pl.ds(tbase, NPT)])


_spmm_call = pl.kernel(
    _spmm_body,
    out_type=jax.ShapeDtypeStruct((NC, NPAD, F), _f32),
    mesh=_mesh,
    compiler_params=_sc_params,
    scratch_types=[
        pltpu.VMEM((SLAB, KCH), jnp.int32),
        pltpu.VMEM((SLAB, KCH), jnp.int32),
        pltpu.VMEM((SLAB, KCH), _f32),
        pltpu.VMEM((2, KCH, F), _f32),
        pltpu.VMEM((SLAB, F), _f32),
        pltpu.VMEM_SHARED((NPAD, F), _f32),
        pltpu.SemaphoreType.DMA,
        pltpu.SemaphoreType.DMA,
    ],
)


RB = 1280


def _sum_body(p_ref, o_ref):
    o_ref[...] = p_ref[0] + p_ref[1]


def _sum_call(p):
    return pl.pallas_call(
        _sum_body,
        grid=(NPAD // RB,),
        in_specs=[pl.BlockSpec((NC, RB, F), lambda i: (0, i, 0))],
        out_specs=pl.BlockSpec((RB, F), lambda i: (i, 0)),
        out_shape=jax.ShapeDtypeStruct((NPAD, F), _f32),
    )(p)


_BN_INV = 1.0 / (1.0 + EPS) ** 0.5


def _comb_mid_body(h_ref, u1_ref, u2p_ref, w_ref, b_ref, bnw_ref, bnb_ref, o_ref):
    h = h_ref[...]
    u2 = u2p_ref[0] + u2p_ref[1]
    w0, w1, w2 = w_ref[0], w_ref[1], w_ref[2]
    z = jnp.dot(h, w0 - w2, preferred_element_type=_f32)
    z = z + jnp.dot(u1_ref[...], w1, preferred_element_type=_f32)
    z = z + 2.0 * jnp.dot(u2, w2, preferred_element_type=_f32)
    z = z + b_ref[...]
    o_ref[...] = jnp.maximum(z, 0.0) * (bnw_ref[...] * _BN_INV) + bnb_ref[...]


def _comb_mid(h, u1, u2p, W, b, bnw, bnb):
    return pl.pallas_call(
        _comb_mid_body,
        grid=(NPAD // RB,),
        in_specs=[
            pl.BlockSpec((RB, F), lambda i: (i, 0)),
            pl.BlockSpec((RB, F), lambda i: (i, 0)),
            pl.BlockSpec((NC, RB, F), lambda i: (0, i, 0)),
            pl.BlockSpec((3, F, F), lambda i: (0, 0, 0)),
            pl.BlockSpec((1, F), lambda i: (0, 0)),
            pl.BlockSpec((1, F), lambda i: (0, 0)),
            pl.BlockSpec((1, F), lambda i: (0, 0)),
        ],
        out_specs=pl.BlockSpec((RB, F), lambda i: (i, 0)),
        out_shape=jax.ShapeDtypeStruct((NPAD, F), _f32),
    )(h, u1, u2p, W, b, bnw, bnb)


def _comb_fin_body(h_ref, u1_ref, u2p_ref, w_ref, b_ref, bnw_ref, bnb_ref,
                   lw_ref, lb_ref, o_ref):
    h = h_ref[...]
    u2 = u2p_ref[0] + u2p_ref[1]
    w0, w1, w2 = w_ref[0], w_ref[1], w_ref[2]
    z = jnp.dot(h, w0 - w2, preferred_element_type=_f32)
    z = z + jnp.dot(u1_ref[...], w1, preferred_element_type=_f32)
    z = z + 2.0 * jnp.dot(u2, w2, preferred_element_type=_f32)
    z = z + b_ref[...]
    h2 = jnp.maximum(z, 0.0) * (bnw_ref[...] * _BN_INV) + bnb_ref[...]
    o_ref[...] = jnp.dot(h2, lw_ref[...], preferred_element_type=_f32) + lb_ref[...]


def _comb_fin(h, u1, u2p, W, b, bnw, bnb, lwT, lb):
    return pl.pallas_call(
        _comb_fin_body,
        grid=(NPAD // RB,),
        in_specs=[
            pl.BlockSpec((RB, F), lambda i: (i, 0)),
            pl.BlockSpec((RB, F), lambda i: (i, 0)),
            pl.BlockSpec((NC, RB, F), lambda i: (0, i, 0)),
            pl.BlockSpec((3, F, F), lambda i: (0, 0, 0)),
            pl.BlockSpec((1, F), lambda i: (0, 0)),
            pl.BlockSpec((1, F), lambda i: (0, 0)),
            pl.BlockSpec((1, F), lambda i: (0, 0)),
            pl.BlockSpec((F, OUT_F), lambda i: (0, 0)),
            pl.BlockSpec((1, OUT_F), lambda i: (0, 0)),
        ],
        out_specs=pl.BlockSpec((RB, OUT_F), lambda i: (i, 0)),
        out_shape=jax.ShapeDtypeStruct((NPAD, OUT_F), _f32),
    )(h, u1, u2p, W, b, bnw, bnb, lwT, lb)


def kernel(x, edge_index, edge_weight, W1, b1, bn1_w, bn1_b, W2, b2, bn2_w, bn2_b, lin_w, lin_b):
    src = jnp.pad(edge_index[0], (0, EPAD - E)).reshape(ROWS, KCH)
    dst = jnp.pad(edge_index[1], (0, EPAD - E)).reshape(ROWS, KCH)
    ew = jnp.pad(edge_weight, (0, EPAD - E)).reshape(ROWS, KCH)
    xp = jnp.pad(x, ((0, NPAD - N), (0, 0)))

    deg0, deg1 = _deg_call(src, ew)
    dinv = _dinv_call(deg0, deg1)
    wn = _wn_call(dinv, src, dst, ew)

    def spmm2(h):
        u1p = _spmm_call(h, src, dst, wn)
        u1 = _sum_call(u1p)
        u2p = _spmm_call(u1, src, dst, wn)
        return u1, u2p

    u1, u2p = spmm2(xp)
    h2 = _comb_mid(xp, u1, u2p, W1, b1.reshape(1, F), bn1_w.reshape(1, F),
                   bn1_b.reshape(1, F))
    v1, v2p = spmm2(h2)
    out = _comb_fin(h2, v1, v2p, W2, b2.reshape(1, F), bn2_w.reshape(1, F),
                    bn2_b.reshape(1, F), lin_w.T, lin_b.reshape(1, OUT_F))
    return out[:N]


# speedup vs baseline: 1.4786x; 1.0938x over previous
import functools

import jax
import jax.numpy as jnp
from jax import lax
from jax.experimental import pallas as pl
from jax.experimental.pallas import tpu as pltpu
from jax.experimental.pallas import tpu_sc as plsc

NC, NS, L = 2, 16, 16
NW = NC * NS
N = 10000
NPAD = 10240
E = 320000
F = 128
KCH = 80
EPAD = 327680
ROWS = EPAD // KCH
RPW = ROWS // NW
NPT = NPAD // NS
EPS = 1e-5
OUT_F = 16

_mesh = plsc.VectorSubcoreMesh(core_axis_name="c", subcore_axis_name="s")
_sc_params = pltpu.CompilerParams(needs_layout_passes=False)

_f32 = jnp.float32


def _ids():
    c = lax.axis_index("c")
    s = lax.axis_index("s")
    return c, s, s * NC + c


def _deg_body(src_hbm, ew_hbm, out0_hbm, out1_hbm, srcb, ewb, zb, dacc):
    c, s, wid = _ids()
    tbase = s * NPT

    @pl.loop(0, NPT // L)
    def _zero(i):
        zb[pl.ds(i * L, L)] = jnp.zeros((L,), _f32)

    pltpu.sync_copy(zb, dacc.at[pl.ds(tbase, NPT)])
    plsc.subcore_barrier()

    pltpu.sync_copy(src_hbm.at[pl.ds(wid * RPW, RPW)], srcb)
    pltpu.sync_copy(ew_hbm.at[pl.ds(wid * RPW, RPW)], ewb)

    @pl.loop(0, RPW)
    def _chunk(jj):
        pltpu.sync_copy(ewb.at[jj], dacc.at[srcb.at[jj]], add=True)

    plsc.subcore_barrier()

    @pl.when(c == 0)
    def _w0():
        pltpu.sync_copy(dacc.at[pl.ds(tbase, NPT)], out0_hbm.at[pl.ds(tbase, NPT)])

    @pl.when(c == 1)
    def _w1():
        pltpu.sync_copy(dacc.at[pl.ds(tbase, NPT)], out1_hbm.at[pl.ds(tbase, NPT)])


_deg_call = pl.kernel(
    _deg_body,
    out_type=[jax.ShapeDtypeStruct((NPAD,), _f32),
              jax.ShapeDtypeStruct((NPAD,), _f32)],
    mesh=_mesh,
    compiler_params=_sc_params,
    scratch_types=[
        pltpu.VMEM((RPW, KCH), jnp.int32),
        pltpu.VMEM((RPW, KCH), _f32),
        pltpu.VMEM((NPT,), _f32),
        pltpu.VMEM_SHARED((NPAD,), _f32),
    ],
)


def _dinv_body(d0_ref, d1_ref, o_ref):
    d = d0_ref[...] + d1_ref[...]
    o_ref[...] = jnp.where(d > 0, lax.rsqrt(jnp.maximum(d, 1e-30)), 0.0)


def _dinv_call(d0, d1):
    r = pl.pallas_call(
        _dinv_body,
        out_shape=jax.ShapeDtypeStruct((NPAD // F, F), _f32),
    )(d0.reshape(NPAD // F, F), d1.reshape(NPAD // F, F))
    return r.reshape(NPAD)


def _wn_body(dinv_hbm, src_hbm, dst_hbm, ew_hbm, wn_hbm,
             dv, srcb, dstb, ewb, wnb):
    c, s, wid = _ids()
    pltpu.sync_copy(dinv_hbm, dv)

    pltpu.sync_copy(src_hbm.at[pl.ds(wid * RPW, RPW)], srcb)
    pltpu.sync_copy(dst_hbm.at[pl.ds(wid * RPW, RPW)], dstb)
    pltpu.sync_copy(ew_hbm.at[pl.ds(wid * RPW, RPW)], ewb)

    @pl.loop(0, RPW)
    def _row(jj):
        for q in range(KCH // L):
            sl = pl.ds(q * L, L)
            g1 = plsc.load_gather(dv, [srcb[jj, sl]])
            g2 = plsc.load_gather(dv, [dstb[jj, sl]])
            wnb[jj, sl] = -(g1 * ewb[jj, sl] * g2)

    pltpu.sync_copy(wnb, wn_hbm.at[pl.ds(wid * RPW, RPW)])


_wn_call = pl.kernel(
    _wn_body,
    out_type=jax.ShapeDtypeStruct((ROWS, KCH), _f32),
    mesh=_mesh,
    compiler_params=_sc_params,
    scratch_types=[
        pltpu.VMEM((NPAD,), _f32),
        pltpu.VMEM((RPW, KCH), jnp.int32),
        pltpu.VMEM((RPW, KCH), jnp.int32),
        pltpu.VMEM((RPW, KCH), _f32),
        pltpu.VMEM((RPW, KCH), _f32),
    ],
)


SLAB = 16


def _scale_chunk(wnb, rbb, jj):
    @pl.loop(0, KCH // L)
    def _scale(g):
        w16 = wnb[jj, pl.ds(g * L, L)]
        for e in range(L):
            wv = w16[e]
            row = g * L + e
            for q in range(F // L):
                sl = pl.ds(q * L, L)
                rbb[row, sl] = rbb[row, sl] * wv


RPW0 = 224
RPW1 = 2 * RPW - RPW0


def _spmm_edges(h_hbm, src_hbm, dst_hbm, wn_hbm, srcb, dstb, wnb, rb, acc,
                sems, base, npieces):
    @pl.loop(0, npieces)
    def _piece(p):
        poff = pl.multiple_of(base + p * SLAB, SLAB)
        pltpu.sync_copy(src_hbm.at[pl.ds(poff, SLAB)], srcb)
        pltpu.sync_copy(dst_hbm.at[pl.ds(poff, SLAB)], dstb)
        pltpu.sync_copy(wn_hbm.at[pl.ds(poff, SLAB)], wnb)

        pltpu.async_copy(h_hbm.at[srcb.at[0]], rb.at[0], sems[0])
        pltpu.async_copy(h_hbm.at[srcb.at[1]], rb.at[1], sems[1])

        @pl.loop(0, SLAB, step=2)
        def _pair(jj):
            for b in range(2):
                jb = jj + b
                pltpu.make_async_copy(h_hbm.at[srcb.at[jb]], rb.at[b], sems[b]).wait()
                _scale_chunk(wnb, rb.at[b], jb)
                pltpu.sync_copy(rb.at[b], acc.at[dstb.at[jb]], add=True)

                @pl.when(jb + 2 < SLAB)
                def _next():
                    pltpu.async_copy(h_hbm.at[srcb.at[jb + 2]], rb.at[b], sems[b])


def _spmm_body(h_hbm, src_hbm, dst_hbm, wn_hbm, out_hbm,
               srcb, dstb, wnb, rb, zb, acc, gs0, gs1):
    c, s, wid = _ids()
    tbase = s * NPT
    sems = (gs0, gs1)

    @pl.loop(0, SLAB)
    def _zero(r):
        for q in range(F // L):
            zb[r, pl.ds(q * L, L)] = jnp.zeros((L,), _f32)

    @pl.loop(0, NPT // SLAB)
    def _zc(b):
        pltpu.sync_copy(zb, acc.at[pl.ds(tbase + b * SLAB, SLAB)])

    plsc.subcore_barrier()

    args = (h_hbm, src_hbm, dst_hbm, wn_hbm, srcb, dstb, wnb, rb, acc, sems)

    @pl.when(c == 0)
    def _c0():
        _spmm_edges(*args, s * RPW0, RPW0 // SLAB)

    @pl.when(c == 1)
    def _c1():
        _spmm_edges(*args, NS * RPW0 + s * RPW1, RPW1 // SLAB)

    plsc.subcore_barrier()
    pltpu.sync_copy(acc.at[pl.ds(tbase, NPT)], out_hbm.at[c, pl.ds(tbase, NPT)])


_spmm_call = pl.kernel(
    _spmm_body,
    out_type=jax.ShapeDtypeStruct((NC, NPAD, F), _f32),
    mesh=_mesh,
    compiler_params=_sc_params,
    scratch_types=[
        pltpu.VMEM((SLAB, KCH), jnp.int32),
        pltpu.VMEM((SLAB, KCH), jnp.int32),
        pltpu.VMEM((SLAB, KCH), _f32),
        pltpu.VMEM((2, KCH, F), _f32),
        pltpu.VMEM((SLAB, F), _f32),
        pltpu.VMEM_SHARED((NPAD, F), _f32),
        pltpu.SemaphoreType.DMA,
        pltpu.SemaphoreType.DMA,
    ],
)


RB = 1280


def _sum_body(p_ref, o_ref):
    o_ref[...] = p_ref[0] + p_ref[1]


def _sum_call(p):
    return pl.pallas_call(
        _sum_body,
        grid=(NPAD // RB,),
        in_specs=[pl.BlockSpec((NC, RB, F), lambda i: (0, i, 0))],
        out_specs=pl.BlockSpec((RB, F), lambda i: (i, 0)),
        out_shape=jax.ShapeDtypeStruct((NPAD, F), _f32),
    )(p)


_BN_INV = 1.0 / (1.0 + EPS) ** 0.5


def _comb_mid_body(h_ref, u1_ref, u2p_ref, w_ref, b_ref, bnw_ref, bnb_ref, o_ref):
    h = h_ref[...]
    u2 = u2p_ref[0] + u2p_ref[1]
    w0, w1, w2 = w_ref[0], w_ref[1], w_ref[2]
    z = jnp.dot(h, w0 - w2, preferred_element_type=_f32)
    z = z + jnp.dot(u1_ref[...], w1, preferred_element_type=_f32)
    z = z + 2.0 * jnp.dot(u2, w2, preferred_element_type=_f32)
    z = z + b_ref[...]
    o_ref[...] = jnp.maximum(z, 0.0) * (bnw_ref[...] * _BN_INV) + bnb_ref[...]


def _comb_mid(h, u1, u2p, W, b, bnw, bnb):
    return pl.pallas_call(
        _comb_mid_body,
        grid=(NPAD // RB,),
        in_specs=[
            pl.BlockSpec((RB, F), lambda i: (i, 0)),
            pl.BlockSpec((RB, F), lambda i: (i, 0)),
            pl.BlockSpec((NC, RB, F), lambda i: (0, i, 0)),
            pl.BlockSpec((3, F, F), lambda i: (0, 0, 0)),
            pl.BlockSpec((1, F), lambda i: (0, 0)),
            pl.BlockSpec((1, F), lambda i: (0, 0)),
            pl.BlockSpec((1, F), lambda i: (0, 0)),
        ],
        out_specs=pl.BlockSpec((RB, F), lambda i: (i, 0)),
        out_shape=jax.ShapeDtypeStruct((NPAD, F), _f32),
    )(h, u1, u2p, W, b, bnw, bnb)


def _comb_fin_body(h_ref, u1_ref, u2p_ref, w_ref, b_ref, bnw_ref, bnb_ref,
                   lw_ref, lb_ref, o_ref):
    h = h_ref[...]
    u2 = u2p_ref[0] + u2p_ref[1]
    w0, w1, w2 = w_ref[0], w_ref[1], w_ref[2]
    z = jnp.dot(h, w0 - w2, preferred_element_type=_f32)
    z = z + jnp.dot(u1_ref[...], w1, preferred_element_type=_f32)
    z = z + 2.0 * jnp.dot(u2, w2, preferred_element_type=_f32)
    z = z + b_ref[...]
    h2 = jnp.maximum(z, 0.0) * (bnw_ref[...] * _BN_INV) + bnb_ref[...]
    o_ref[...] = jnp.dot(h2, lw_ref[...], preferred_element_type=_f32) + lb_ref[...]


def _comb_fin(h, u1, u2p, W, b, bnw, bnb, lwT, lb):
    return pl.pallas_call(
        _comb_fin_body,
        grid=(NPAD // RB,),
        in_specs=[
            pl.BlockSpec((RB, F), lambda i: (i, 0)),
            pl.BlockSpec((RB, F), lambda i: (i, 0)),
            pl.BlockSpec((NC, RB, F), lambda i: (0, i, 0)),
            pl.BlockSpec((3, F, F), lambda i: (0, 0, 0)),
            pl.BlockSpec((1, F), lambda i: (0, 0)),
            pl.BlockSpec((1, F), lambda i: (0, 0)),
            pl.BlockSpec((1, F), lambda i: (0, 0)),
            pl.BlockSpec((F, OUT_F), lambda i: (0, 0)),
            pl.BlockSpec((1, OUT_F), lambda i: (0, 0)),
        ],
        out_specs=pl.BlockSpec((RB, OUT_F), lambda i: (i, 0)),
        out_shape=jax.ShapeDtypeStruct((NPAD, OUT_F), _f32),
    )(h, u1, u2p, W, b, bnw, bnb, lwT, lb)


def kernel(x, edge_index, edge_weight, W1, b1, bn1_w, bn1_b, W2, b2, bn2_w, bn2_b, lin_w, lin_b):
    src = jnp.pad(edge_index[0], (0, EPAD - E)).reshape(ROWS, KCH)
    dst = jnp.pad(edge_index[1], (0, EPAD - E)).reshape(ROWS, KCH)
    ew = jnp.pad(edge_weight, (0, EPAD - E)).reshape(ROWS, KCH)
    xp = jnp.pad(x, ((0, NPAD - N), (0, 0)))

    deg0, deg1 = _deg_call(src, ew)
    dinv = _dinv_call(deg0, deg1)
    wn = _wn_call(dinv, src, dst, ew)

    def spmm2(h):
        u1p = _spmm_call(h, src, dst, wn)
        u1 = _sum_call(u1p)
        u2p = _spmm_call(u1, src, dst, wn)
        return u1, u2p

    u1, u2p = spmm2(xp)
    h2 = _comb_mid(xp, u1, u2p, W1, b1.reshape(1, F), bn1_w.reshape(1, F),
                   bn1_b.reshape(1, F))
    v1, v2p = spmm2(h2)
    out = _comb_fin(h2, v1, v2p, W2, b2.reshape(1, F), bn2_w.reshape(1, F),
                    bn2_b.reshape(1, F), lin_w.T, lin_b.reshape(1, OUT_F))
    return out[:N]


# speedup vs baseline: 1.4845x; 1.0040x over previous
import functools

import jax
import jax.numpy as jnp
from jax import lax
from jax.experimental import pallas as pl
from jax.experimental.pallas import tpu as pltpu
from jax.experimental.pallas import tpu_sc as plsc

NC, NS, L = 2, 16, 16
NW = NC * NS
N = 10000
NPAD = 10240
E = 320000
F = 128
KCH = 80
EPAD = 327680
ROWS = EPAD // KCH
RPW = ROWS // NW
NPT = NPAD // NS
EPS = 1e-5
OUT_F = 16

_mesh = plsc.VectorSubcoreMesh(core_axis_name="c", subcore_axis_name="s")
_sc_params = pltpu.CompilerParams(needs_layout_passes=False)

_f32 = jnp.float32


def _ids():
    c = lax.axis_index("c")
    s = lax.axis_index("s")
    return c, s, s * NC + c


def _deg_body(src_hbm, ew_hbm, out0_hbm, out1_hbm, srcb, ewb, zb, dacc):
    c, s, wid = _ids()
    tbase = s * NPT

    @pl.loop(0, NPT // L)
    def _zero(i):
        zb[pl.ds(i * L, L)] = jnp.zeros((L,), _f32)

    pltpu.sync_copy(zb, dacc.at[pl.ds(tbase, NPT)])
    plsc.subcore_barrier()

    pltpu.sync_copy(src_hbm.at[pl.ds(wid * RPW, RPW)], srcb)
    pltpu.sync_copy(ew_hbm.at[pl.ds(wid * RPW, RPW)], ewb)

    @pl.loop(0, RPW)
    def _chunk(jj):
        pltpu.sync_copy(ewb.at[jj], dacc.at[srcb.at[jj]], add=True)

    plsc.subcore_barrier()

    @pl.when(c == 0)
    def _w0():
        pltpu.sync_copy(dacc.at[pl.ds(tbase, NPT)], out0_hbm.at[pl.ds(tbase, NPT)])

    @pl.when(c == 1)
    def _w1():
        pltpu.sync_copy(dacc.at[pl.ds(tbase, NPT)], out1_hbm.at[pl.ds(tbase, NPT)])


_deg_call = pl.kernel(
    _deg_body,
    out_type=[jax.ShapeDtypeStruct((NPAD,), _f32),
              jax.ShapeDtypeStruct((NPAD,), _f32)],
    mesh=_mesh,
    compiler_params=_sc_params,
    scratch_types=[
        pltpu.VMEM((RPW, KCH), jnp.int32),
        pltpu.VMEM((RPW, KCH), _f32),
        pltpu.VMEM((NPT,), _f32),
        pltpu.VMEM_SHARED((NPAD,), _f32),
    ],
)


def _dinv_body(d0_ref, d1_ref, o_ref):
    d = d0_ref[...] + d1_ref[...]
    o_ref[...] = jnp.where(d > 0, lax.rsqrt(jnp.maximum(d, 1e-30)), 0.0)


def _dinv_call(d0, d1):
    r = pl.pallas_call(
        _dinv_body,
        out_shape=jax.ShapeDtypeStruct((NPAD // F, F), _f32),
    )(d0.reshape(NPAD // F, F), d1.reshape(NPAD // F, F))
    return r.reshape(NPAD)


def _wn_body(dinv_hbm, src_hbm, dst_hbm, ew_hbm, wn_hbm,
             dv, srcb, dstb, ewb, wnb):
    c, s, wid = _ids()
    pltpu.sync_copy(dinv_hbm, dv)

    pltpu.sync_copy(src_hbm.at[pl.ds(wid * RPW, RPW)], srcb)
    pltpu.sync_copy(dst_hbm.at[pl.ds(wid * RPW, RPW)], dstb)
    pltpu.sync_copy(ew_hbm.at[pl.ds(wid * RPW, RPW)], ewb)

    @pl.loop(0, RPW)
    def _row(jj):
        for q in range(KCH // L):
            sl = pl.ds(q * L, L)
            g1 = plsc.load_gather(dv, [srcb[jj, sl]])
            g2 = plsc.load_gather(dv, [dstb[jj, sl]])
            wnb[jj, sl] = -(g1 * ewb[jj, sl] * g2)

    pltpu.sync_copy(wnb, wn_hbm.at[pl.ds(wid * RPW, RPW)])


_wn_call = pl.kernel(
    _wn_body,
    out_type=jax.ShapeDtypeStruct((ROWS, KCH), _f32),
    mesh=_mesh,
    compiler_params=_sc_params,
    scratch_types=[
        pltpu.VMEM((NPAD,), _f32),
        pltpu.VMEM((RPW, KCH), jnp.int32),
        pltpu.VMEM((RPW, KCH), jnp.int32),
        pltpu.VMEM((RPW, KCH), _f32),
        pltpu.VMEM((RPW, KCH), _f32),
    ],
)


SLAB = 16


def _scale_chunk(wnb, rbb, jj):
    @pl.loop(0, KCH // L)
    def _scale(g):
        w16 = wnb[jj, pl.ds(g * L, L)]
        for e in range(L):
            wv = w16[e]
            row = g * L + e
            for q in range(F // L):
                sl = pl.ds(q * L, L)
                rbb[row, sl] = rbb[row, sl] * wv


RPW0 = 240
RPW1 = 2 * RPW - RPW0


def _spmm_edges(h_hbm, src_hbm, dst_hbm, wn_hbm, srcb, dstb, wnb, rb, acc,
                sems, base, npieces):
    @pl.loop(0, npieces)
    def _piece(p):
        poff = pl.multiple_of(base + p * SLAB, SLAB)
        pltpu.sync_copy(src_hbm.at[pl.ds(poff, SLAB)], srcb)
        pltpu.sync_copy(dst_hbm.at[pl.ds(poff, SLAB)], dstb)
        pltpu.sync_copy(wn_hbm.at[pl.ds(poff, SLAB)], wnb)

        pltpu.async_copy(h_hbm.at[srcb.at[0]], rb.at[0], sems[0])
        pltpu.async_copy(h_hbm.at[srcb.at[1]], rb.at[1], sems[1])

        @pl.loop(0, SLAB, step=2)
        def _pair(jj):
            for b in range(2):
                jb = jj + b
                pltpu.make_async_copy(h_hbm.at[srcb.at[jb]], rb.at[b], sems[b]).wait()
                _scale_chunk(wnb, rb.at[b], jb)
                pltpu.sync_copy(rb.at[b], acc.at[dstb.at[jb]], add=True)

                @pl.when(jb + 2 < SLAB)
                def _next():
                    pltpu.async_copy(h_hbm.at[srcb.at[jb + 2]], rb.at[b], sems[b])


def _spmm_body(h_hbm, src_hbm, dst_hbm, wn_hbm, out_hbm,
               srcb, dstb, wnb, rb, zb, acc, gs0, gs1):
    c, s, wid = _ids()
    tbase = s * NPT
    sems = (gs0, gs1)

    @pl.loop(0, SLAB)
    def _zero(r):
        for q in range(F // L):
            zb[r, pl.ds(q * L, L)] = jnp.zeros((L,), _f32)

    @pl.loop(0, NPT // SLAB)
    def _zc(b):
        pltpu.sync_copy(zb, acc.at[pl.ds(tbase + b * SLAB, SLAB)])

    plsc.subcore_barrier()

    args = (h_hbm, src_hbm, dst_hbm, wn_hbm, srcb, dstb, wnb, rb, acc, sems)

    @pl.when(c == 0)
    def _c0():
        _spmm_edges(*args, s * RPW0, RPW0 // SLAB)

    @pl.when(c == 1)
    def _c1():
        _spmm_edges(*args, NS * RPW0 + s * RPW1, RPW1 // SLAB)

    plsc.subcore_barrier()
    pltpu.sync_copy(acc.at[pl.ds(tbase, NPT)], out_hbm.at[c, pl.ds(tbase, NPT)])


_spmm_call = pl.kernel(
    _spmm_body,
    out_type=jax.ShapeDtypeStruct((NC, NPAD, F), _f32),
    mesh=_mesh,
    compiler_params=_sc_params,
    scratch_types=[
        pltpu.VMEM((SLAB, KCH), jnp.int32),
        pltpu.VMEM((SLAB, KCH), jnp.int32),
        pltpu.VMEM((SLAB, KCH), _f32),
        pltpu.VMEM((2, KCH, F), _f32),
        pltpu.VMEM((SLAB, F), _f32),
        pltpu.VMEM_SHARED((NPAD, F), _f32),
        pltpu.SemaphoreType.DMA,
        pltpu.SemaphoreType.DMA,
    ],
)


RB = 1280


def _sum_body(p_ref, o_ref):
    o_ref[...] = p_ref[0] + p_ref[1]


def _sum_call(p):
    return pl.pallas_call(
        _sum_body,
        grid=(NPAD // RB,),
        in_specs=[pl.BlockSpec((NC, RB, F), lambda i: (0, i, 0))],
        out_specs=pl.BlockSpec((RB, F), lambda i: (i, 0)),
        out_shape=jax.ShapeDtypeStruct((NPAD, F), _f32),
    )(p)


_BN_INV = 1.0 / (1.0 + EPS) ** 0.5


def _comb_mid_body(h_ref, u1_ref, u2p_ref, w_ref, b_ref, bnw_ref, bnb_ref, o_ref):
    h = h_ref[...]
    u2 = u2p_ref[0] + u2p_ref[1]
    w0, w1, w2 = w_ref[0], w_ref[1], w_ref[2]
    z = jnp.dot(h, w0 - w2, preferred_element_type=_f32)
    z = z + jnp.dot(u1_ref[...], w1, preferred_element_type=_f32)
    z = z + 2.0 * jnp.dot(u2, w2, preferred_element_type=_f32)
    z = z + b_ref[...]
    o_ref[...] = jnp.maximum(z, 0.0) * (bnw_ref[...] * _BN_INV) + bnb_ref[...]


def _comb_mid(h, u1, u2p, W, b, bnw, bnb):
    return pl.pallas_call(
        _comb_mid_body,
        grid=(NPAD // RB,),
        in_specs=[
            pl.BlockSpec((RB, F), lambda i: (i, 0)),
            pl.BlockSpec((RB, F), lambda i: (i, 0)),
            pl.BlockSpec((NC, RB, F), lambda i: (0, i, 0)),
            pl.BlockSpec((3, F, F), lambda i: (0, 0, 0)),
            pl.BlockSpec((1, F), lambda i: (0, 0)),
            pl.BlockSpec((1, F), lambda i: (0, 0)),
            pl.BlockSpec((1, F), lambda i: (0, 0)),
        ],
        out_specs=pl.BlockSpec((RB, F), lambda i: (i, 0)),
        out_shape=jax.ShapeDtypeStruct((NPAD, F), _f32),
    )(h, u1, u2p, W, b, bnw, bnb)


def _comb_fin_body(h_ref, u1_ref, u2p_ref, w_ref, b_ref, bnw_ref, bnb_ref,
                   lw_ref, lb_ref, o_ref):
    h = h_ref[...]
    u2 = u2p_ref[0] + u2p_ref[1]
    w0, w1, w2 = w_ref[0], w_ref[1], w_ref[2]
    z = jnp.dot(h, w0 - w2, preferred_element_type=_f32)
    z = z + jnp.dot(u1_ref[...], w1, preferred_element_type=_f32)
    z = z + 2.0 * jnp.dot(u2, w2, preferred_element_type=_f32)
    z = z + b_ref[...]
    h2 = jnp.maximum(z, 0.0) * (bnw_ref[...] * _BN_INV) + bnb_ref[...]
    o_ref[...] = jnp.dot(h2, lw_ref[...], preferred_element_type=_f32) + lb_ref[...]


def _comb_fin(h, u1, u2p, W, b, bnw, bnb, lwT, lb):
    return pl.pallas_call(
        _comb_fin_body,
        grid=(NPAD // RB,),
        in_specs=[
            pl.BlockSpec((RB, F), lambda i: (i, 0)),
            pl.BlockSpec((RB, F), lambda i: (i, 0)),
            pl.BlockSpec((NC, RB, F), lambda i: (0, i, 0)),
            pl.BlockSpec((3, F, F), lambda i: (0, 0, 0)),
            pl.BlockSpec((1, F), lambda i: (0, 0)),
            pl.BlockSpec((1, F), lambda i: (0, 0)),
            pl.BlockSpec((1, F), lambda i: (0, 0)),
            pl.BlockSpec((F, OUT_F), lambda i: (0, 0)),
            pl.BlockSpec((1, OUT_F), lambda i: (0, 0)),
        ],
        out_specs=pl.BlockSpec((RB, OUT_F), lambda i: (i, 0)),
        out_shape=jax.ShapeDtypeStruct((NPAD, OUT_F), _f32),
    )(h, u1, u2p, W, b, bnw, bnb, lwT, lb)


def kernel(x, edge_index, edge_weight, W1, b1, bn1_w, bn1_b, W2, b2, bn2_w, bn2_b, lin_w, lin_b):
    src = jnp.pad(edge_index[0], (0, EPAD - E)).reshape(ROWS, KCH)
    dst = jnp.pad(edge_index[1], (0, EPAD - E)).reshape(ROWS, KCH)
    ew = jnp.pad(edge_weight, (0, EPAD - E)).reshape(ROWS, KCH)
    xp = jnp.pad(x, ((0, NPAD - N), (0, 0)))

    deg0, deg1 = _deg_call(src, ew)
    dinv = _dinv_call(deg0, deg1)
    wn = _wn_call(dinv, src, dst, ew)

    def spmm2(h):
        u1p = _spmm_call(h, src, dst, wn)
        u1 = _sum_call(u1p)
        u2p = _spmm_call(u1, src, dst, wn)
        return u1, u2p

    u1, u2p = spmm2(xp)
    h2 = _comb_mid(xp, u1, u2p, W1, b1.reshape(1, F), bn1_w.reshape(1, F),
                   bn1_b.reshape(1, F))
    v1, v2p = spmm2(h2)
    out = _comb_fin(h2, v1, v2p, W2, b2.reshape(1, F), bn2_w.reshape(1, F),
                    bn2_b.reshape(1, F), lin_w.T, lin_b.reshape(1, OUT_F))
    return out[:N]
